# baseline (speedup 1.0000x reference)
import jax
import jax.numpy as jnp
from jax.experimental import pallas as pl

N = 2048
BR = 128
NBLK = N // BR
F1, F2, F3 = 384, 640, 1152


def _dot(a, b):
    return jnp.dot(a, b, preferred_element_type=jnp.float32)


def _pad2(w, r, c):
    return jnp.pad(w, ((0, r - w.shape[0]), (0, c - w.shape[1])))


def _pad_row(b, c):
    return jnp.pad(b, (0, c - b.shape[0])).reshape(1, c)


def _pad_col(p, r):
    return jnp.pad(p, (0, r - p.shape[0])).reshape(r, 1)


def _emb_body(c_ref, cm_ref, t_ref,
              tw0a, tb0a, tw0b, tb0b, cw0a, cb0a, cw0b, cb0b,
              tw1a, tb1a, tw1b, tb1b, cw1a, cb1a, cw1b, cb1b,
              tw2a, tb2a, tw2b, tb2b, cw2a, cb2a, cw2b, cb2b,
              t0_o, c0_o, t1_o, c1_o, t2_o, c2_o):
    cmv = 1.0 - cm_ref[0, 0]
    cb = c_ref[...] * cmv
    tv = t_ref[0, 0]

    def mlp_t(wa, ba, wb, bb):
        h = jax.nn.gelu(tv * wa[...] + ba[...])
        return _dot(h, wb[...]) + bb[...]

    def mlp_c(wa, ba, wb, bb):
        h = jax.nn.gelu(_dot(cb, wa[...]) + ba[...])
        return _dot(h, wb[...]) + bb[...]

    t0_o[...] = mlp_t(tw0a, tb0a, tw0b, tb0b)
    c0_o[...] = mlp_c(cw0a, cb0a, cw0b, cb0b)
    t1_o[...] = mlp_t(tw1a, tb1a, tw1b, tb1b)
    c1_o[...] = mlp_c(cw1a, cb1a, cw1b, cb1b)
    t2_o[...] = mlp_t(tw2a, tb2a, tw2b, tb2b)
    c2_o[...] = mlp_c(cw2a, cb2a, cw2b, cb2b)


def _dinv_body(a_ref, o_ref):
    o_ref[...] = jax.lax.rsqrt(jnp.sum(a_ref[...], axis=1, keepdims=True) + 1.0)


def _n0h1_body(a_ref, dvr_ref, dvb_ref, w1_ref, b1_ref, p1_ref,
               n0_ref, h1_ref, s1_ref):
    i = pl.program_id(0)
    a = a_ref[...]
    rows = i * BR + jax.lax.broadcasted_iota(jnp.int32, (BR, N), 0)
    cols = jax.lax.broadcasted_iota(jnp.int32, (BR, N), 1)
    ah = a + jnp.where(rows == cols, 1.0, 0.0)
    n0 = dvb_ref[...] * ah * dvr_ref[...]
    n0_ref[...] = n0
    h = _dot(n0, w1_ref[...]) + b1_ref[...]
    h1_ref[...] = h
    p = p1_ref[...]
    nrm = jnp.maximum(jnp.sqrt(jnp.sum(p * p)), 1e-30)
    s1_ref[...] = _dot(h, p) / nrm


def _make_rank_body(k):
    def _rank_body(s_ref, st_ref, m_ref, mt_ref, mo_ref, g_ref):
        s = s_ref[...]
        sm = jnp.where(m_ref[...] > 0, s, -3.0e38)
        smt = jnp.where(mt_ref[...] > 0, st_ref[...], -3.0e38)
        ridx = jax.lax.broadcasted_iota(jnp.int32, (N, 1), 0)
        r = jnp.zeros((N, 1), jnp.float32)
        ch = 512
        for c in range(N // ch):
            col = smt[:, c * ch:(c + 1) * ch]
            cidx = c * ch + jax.lax.broadcasted_iota(jnp.int32, (1, ch), 1)
            gt = (col > sm) | ((col == sm) & (cidx < ridx))
            r = r + jnp.sum(gt.astype(jnp.float32), axis=1, keepdims=True)
        m = (r < k).astype(jnp.float32)
        mo_ref[...] = m
        g_ref[...] = m * jnp.tanh(s)
    return _rank_body


def _deg_body(a_ref, mf_ref, mb_ref, u_ref):
    mb = mb_ref[...]
    d = _dot(a_ref[...], mf_ref[...]) + mb
    u_ref[...] = mb * jax.lax.rsqrt(d + (1.0 - mb))


def _xw_body(a_ref, h_ref, ce_ref, te_ref, tm_ref, u_ref, w_ref, o_ref):
    x = a_ref[...] * h_ref[...] * ce_ref[...] + tm_ref[...] * te_ref[...]
    o_ref[...] = u_ref[...] * _dot(x, w_ref[...])


def _gcn_body(a_ref, tf_ref, tb_ref, u_ref, b_ref, p_ref, h_ref, s_ref):
    m = _dot(a_ref[...], tf_ref[...]) + tb_ref[...]
    h = u_ref[...] * m + b_ref[...]
    h_ref[...] = h
    p = p_ref[...]
    nrm = jnp.maximum(jnp.sqrt(jnp.sum(p * p)), 1e-30)
    s_ref[...] = _dot(h, p) / nrm


def _fin_body(n0_ref, y_ref, b_ref, o_ref):
    o_ref[...] = _dot(n0_ref[...], y_ref[...]) + b_ref[...]


def _vspec(bm):
    return pl.BlockSpec((bm, 1), lambda i: (i, 0))


def _fix(shape):
    return pl.BlockSpec(shape, lambda i: (0, 0))


def _rowspec(bm, n):
    return pl.BlockSpec((bm, n), lambda i: (i, 0))


def _run_emb(cbp, cm, t, ws):
    dims = [F3, F3, F2, F2, F1, F1]
    outs = tuple(jax.ShapeDtypeStruct((1, d), jnp.float32) for d in dims)
    return pl.pallas_call(
        _emb_body,
        out_shape=outs,
    )(cbp, cm, t, *ws)


def _run_dinv(a):
    return pl.pallas_call(
        _dinv_body,
        grid=(NBLK,),
        in_specs=[_rowspec(BR, N)],
        out_specs=_vspec(BR),
        out_shape=jax.ShapeDtypeStruct((N, 1), jnp.float32),
    )(a)


def _run_n0h1(a, dvr, dv, w1, b1, p1):
    return pl.pallas_call(
        _n0h1_body,
        grid=(NBLK,),
        in_specs=[_rowspec(BR, N), _fix((1, N)), _vspec(BR),
                  _fix((N, F1)), _fix((1, F1)), _fix((F1, 1))],
        out_specs=(_rowspec(BR, N), _rowspec(BR, F1), _vspec(BR)),
        out_shape=(jax.ShapeDtypeStruct((N, N), jnp.float32),
                   jax.ShapeDtypeStruct((N, F1), jnp.float32),
                   jax.ShapeDtypeStruct((N, 1), jnp.float32)),
    )(a, dvr, dv, w1, b1, p1)


def _run_rank(s, mprev, k):
    st = s.reshape(1, N)
    mt = mprev.reshape(1, N)
    return pl.pallas_call(
        _make_rank_body(k),
        out_shape=(jax.ShapeDtypeStruct((N, 1), jnp.float32),
                   jax.ShapeDtypeStruct((N, 1), jnp.float32)),
    )(s, st, mprev, mt)


def _run_deg(a, m):
    return pl.pallas_call(
        _deg_body,
        grid=(NBLK,),
        in_specs=[_rowspec(BR, N), _fix((N, 1)), _vspec(BR)],
        out_specs=_vspec(BR),
        out_shape=jax.ShapeDtypeStruct((N, 1), jnp.float32),
    )(a, m, m)


def _run_xw(avec, h, ce, te, tm, u, w):
    fi, fo = w.shape
    return pl.pallas_call(
        _xw_body,
        grid=(NBLK,),
        in_specs=[_vspec(BR), _rowspec(BR, fi), _fix((1, fi)), _fix((1, fi)),
                  _vspec(BR), _vspec(BR), _fix((fi, fo))],
        out_specs=_rowspec(BR, fo),
        out_shape=jax.ShapeDtypeStruct((N, fo), jnp.float32),
    )(avec, h, ce, te, tm, u, w)


def _run_gcn(a, tmat, u, b, p):
    f = tmat.shape[1]
    return pl.pallas_call(
        _gcn_body,
        grid=(NBLK,),
        in_specs=[_rowspec(BR, N), _fix((N, f)), _rowspec(BR, f),
                  _vspec(BR), _fix((1, f)), _fix((f, 1))],
        out_specs=(_rowspec(BR, f), _vspec(BR)),
        out_shape=(jax.ShapeDtypeStruct((N, f), jnp.float32),
                   jax.ShapeDtypeStruct((N, 1), jnp.float32)),
    )(a, tmat, tmat, u, b, p)


def _run_final(n0, y, b):
    bm = 256
    return pl.pallas_call(
        _fin_body,
        grid=(N // bm,),
        in_specs=[_rowspec(bm, N), _fix((N, N)), _fix((1, N))],
        out_specs=_rowspec(bm, N),
        out_shape=jax.ShapeDtypeStruct((N, N), jnp.float32),
    )(n0, y, b)


def kernel(x, c, t, context_mask, W1, b1, p1, W2, b2, p2, W3, b3, p3,
           Wu1, bu1, Wu2, bu2, Wu3, bu3,
           TW0a, Tb0a, TW0b, Tb0b, CW0a, Cb0a, CW0b, Cb0b,
           TW1a, Tb1a, TW1b, Tb1b, CW1a, Cb1a, CW1b, Cb1b,
           TW2a, Tb2a, TW2b, Tb2b, CW2a, Cb2a, CW2b, Cb2b):
    a0 = x[0, 0]

    w1p = _pad2(W1, N, F1)
    b1p = _pad_row(b1, F1)
    p1p = _pad_col(p1, F1)
    w2p = _pad2(W2, F1, F2)
    b2p = _pad_row(b2, F2)
    p2p = _pad_col(p2, F2)
    w3p = _pad2(W3, F2, F3)
    b3p = _pad_row(b3, F3)
    p3p = _pad_col(p3, F3)
    wu1p = _pad2(Wu1, F3, F2)
    bu1p = _pad_row(bu1, F2)
    wu2p = _pad2(Wu2, F2, F1)
    bu2p = _pad_row(bu2, F1)
    wu3p = _pad2(Wu3, F1, N)
    bu3p = _pad_row(bu3, N)

    cbp = _pad2(c, 1, 16)
    cmp_ = context_mask.reshape(1, 1)
    tp = t.reshape(1, 1)
    embw = [
        _pad2(TW0a, 1, F3), _pad_row(Tb0a, F3), _pad2(TW0b, F3, F3), _pad_row(Tb0b, F3),
        _pad2(CW0a, 16, F3), _pad_row(Cb0a, F3), _pad2(CW0b, F3, F3), _pad_row(Cb0b, F3),
        _pad2(TW1a, 1, F2), _pad_row(Tb1a, F2), _pad2(TW1b, F2, F2), _pad_row(Tb1b, F2),
        _pad2(CW1a, 16, F2), _pad_row(Cb1a, F2), _pad2(CW1b, F2, F2), _pad_row(Cb1b, F2),
        _pad2(TW2a, 1, F1), _pad_row(Tb2a, F1), _pad2(TW2b, F1, F1), _pad_row(Tb2b, F1),
        _pad2(CW2a, 16, F1), _pad_row(Cb2a, F1), _pad2(CW2b, F1, F1), _pad_row(Cb2b, F1),
    ]
    temb0, cemb0, temb1, cemb1, temb2, cemb2 = _run_emb(cbp, cmp_, tp, embw)

    ones_v = jnp.ones((N, 1), jnp.float32)
    ones1 = jnp.ones((1, F1), jnp.float32)
    zeros1 = jnp.zeros((1, F1), jnp.float32)
    zp1 = jnp.zeros((F2, 1), jnp.float32)
    zp2 = jnp.zeros((F1, 1), jnp.float32)

    dv = _run_dinv(a0)
    dvr = dv.reshape(1, N)
    n0, h1, s1 = _run_n0h1(a0, dvr, dv, w1p, b1p, p1p)

    m1, gm1 = _run_rank(s1, ones_v, 1024)
    u1 = _run_deg(a0, m1)
    t2 = _run_xw(gm1, h1, ones1, zeros1, m1, u1, w2p)
    h2, s2 = _run_gcn(a0, t2, u1, b2p, p2p)

    m2, gm2 = _run_rank(s2, m1, 512)
    u2 = _run_deg(a0, m2)
    t3 = _run_xw(gm2, h2, jnp.ones((1, F2), jnp.float32),
                 jnp.zeros((1, F2), jnp.float32), m2, u2, w3p)
    h3, s3 = _run_gcn(a0, t3, u2, b3p, p3p)

    m3, gm3 = _run_rank(s3, m2, 6)

    tu1 = _run_xw(gm3, h3, cemb0, temb0, m3, u2, wu1p)
    x2r, _ = _run_gcn(a0, tu1, u2, bu1p, zp1)

    tu2 = _run_xw(m2, x2r, cemb1, temb1, m2, u1, wu2p)
    x1r, _ = _run_gcn(a0, tu2, u1, bu2p, zp2)

    y = _run_xw(m1, x1r, cemb2, temb2, m1, ones_v, wu3p)
    return _run_final(n0, y, bu3p)


# speedup vs baseline: 1.3774x; 1.3774x over previous
import jax
import jax.numpy as jnp
from jax.experimental import pallas as pl

N = 2048
K2 = 512
BR = 128
NBLK = N // BR
F1, F2, F3 = 384, 640, 1152


def _dot(a, b):
    return jnp.dot(a, b, preferred_element_type=jnp.float32)


def _dotb(a, b):
    return jnp.dot(a.astype(jnp.bfloat16), b.astype(jnp.bfloat16),
                   preferred_element_type=jnp.float32)


def _pad2(w, r, c):
    return jnp.pad(w, ((0, r - w.shape[0]), (0, c - w.shape[1])))


def _pad_row(b, c):
    return jnp.pad(b, (0, c - b.shape[0])).reshape(1, c)


def _pad_col(p, r):
    return jnp.pad(p, (0, r - p.shape[0])).reshape(r, 1)


def _pnorm(p):
    return jnp.maximum(jnp.sqrt(jnp.sum(p * p)), 1e-30)


def _emb_body(c_ref, cm_ref, t_ref,
              tw0a, tb0a, tw0b, tb0b, cw0a, cb0a, cw0b, cb0b,
              tw1a, tb1a, tw1b, tb1b, cw1a, cb1a, cw1b, cb1b,
              tw2a, tb2a, tw2b, tb2b, cw2a, cb2a, cw2b, cb2b,
              t0_o, c0_o, t1_o, c1_o, t2_o, c2_o):
    cmv = 1.0 - cm_ref[0, 0]
    cb = c_ref[...] * cmv
    tv = t_ref[0, 0]

    def mlp_t(wa, ba, wb, bb):
        h = jax.nn.gelu(tv * wa[...] + ba[...])
        return _dot(h, wb[...]) + bb[...]

    def mlp_c(wa, ba, wb, bb):
        h = jax.nn.gelu(_dot(cb, wa[...]) + ba[...])
        return _dot(h, wb[...]) + bb[...]

    t0_o[...] = mlp_t(tw0a, tb0a, tw0b, tb0b)
    c0_o[...] = mlp_c(cw0a, cb0a, cw0b, cb0b)
    t1_o[...] = mlp_t(tw1a, tb1a, tw1b, tb1b)
    c1_o[...] = mlp_c(cw1a, cb1a, cw1b, cb1b)
    t2_o[...] = mlp_t(tw2a, tb2a, tw2b, tb2b)
    c2_o[...] = mlp_c(cw2a, cb2a, cw2b, cb2b)


def _dinv_body(a_ref, o_ref):
    o_ref[...] = jax.lax.rsqrt(jnp.sum(a_ref[...], axis=1, keepdims=True) + 1.0)


def _n0h1_body(a_ref, dvr_ref, dvb_ref, w1_ref, b1_ref, p1_ref,
               n0_ref, h1_ref, s1_ref):
    i = pl.program_id(0)
    a = a_ref[...]
    rows = i * BR + jax.lax.broadcasted_iota(jnp.int32, (BR, N), 0)
    cols = jax.lax.broadcasted_iota(jnp.int32, (BR, N), 1)
    ah = a + jnp.where(rows == cols, 1.0, 0.0)
    n0 = dvb_ref[...] * ah * dvr_ref[...]
    n0_ref[...] = n0
    h = _dot(n0, w1_ref[...]) + b1_ref[...]
    h1_ref[...] = h
    p = p1_ref[...]
    s1_ref[...] = _dot(h, p) / _pnorm(p)


def _rank1_body(s_ref, st_ref, mo_ref, g_ref):
    s = s_ref[...]
    st = st_ref[...]
    ridx = jax.lax.broadcasted_iota(jnp.int32, (N, 1), 0)
    r = jnp.zeros((N, 1), jnp.float32)
    ch = 512
    for c in range(N // ch):
        col = st[:, c * ch:(c + 1) * ch]
        cidx = c * ch + jax.lax.broadcasted_iota(jnp.int32, (1, ch), 1)
        gt = (col > s) | ((col == s) & (cidx < ridx))
        r = r + jnp.sum(gt.astype(jnp.float32), axis=1, keepdims=True)
    m = (r < 1024).astype(jnp.float32)
    mo_ref[...] = m
    g_ref[...] = m * jnp.tanh(s)


def _rank2_body(s_ref, st_ref, m_ref, mt_ref, mo_ref, g_ref, oh_ref, oht_ref):
    s = s_ref[...]
    sm = jnp.where(m_ref[...] > 0, s, -3.0e38)
    smt = jnp.where(mt_ref[...] > 0, st_ref[...], -3.0e38)
    ridx = jax.lax.broadcasted_iota(jnp.int32, (N, 1), 0)
    cidx_f = jax.lax.broadcasted_iota(jnp.int32, (1, N), 1)
    ch = 512
    r = jnp.zeros((N, 1), jnp.float32)
    rt = jnp.zeros((1, N), jnp.float32)
    for c in range(N // ch):
        col = smt[:, c * ch:(c + 1) * ch]
        cidx = c * ch + jax.lax.broadcasted_iota(jnp.int32, (1, ch), 1)
        gt = (col > sm) | ((col == sm) & (cidx < ridx))
        r = r + jnp.sum(gt.astype(jnp.float32), axis=1, keepdims=True)
        rowv = sm[c * ch:(c + 1) * ch, :]
        rix = c * ch + jax.lax.broadcasted_iota(jnp.int32, (ch, 1), 0)
        gt2 = (rowv > smt) | ((rowv == smt) & (rix < cidx_f))
        rt = rt + jnp.sum(gt2.astype(jnp.float32), axis=0, keepdims=True)
    m = (r < K2).astype(jnp.float32)
    mo_ref[...] = m
    g_ref[...] = m * jnp.tanh(s)
    kidx = jax.lax.broadcasted_iota(jnp.int32, (1, K2), 1)
    oh_ref[...] = (r.astype(jnp.int32) == kidx).astype(jnp.float32)
    kidx2 = jax.lax.broadcasted_iota(jnp.int32, (K2, 1), 0)
    oht_ref[...] = (rt.astype(jnp.int32) == kidx2).astype(jnp.float32)


def _deg_body(a_ref, mf_ref, mb_ref, u_ref):
    mb = mb_ref[...]
    d = _dot(a_ref[...], mf_ref[...]) + mb
    u_ref[...] = mb * jax.lax.rsqrt(d + (1.0 - mb))


def _xw_body(a_ref, h_ref, u_ref, w_ref, o_ref):
    x = a_ref[...] * h_ref[...]
    o_ref[...] = u_ref[...] * _dot(x, w_ref[...])


def _gcn_body(a_ref, tf_ref, tb_ref, u_ref, b_ref, p_ref, h_ref, s_ref):
    m = _dot(a_ref[...], tf_ref[...]) + tb_ref[...]
    h = u_ref[...] * m + b_ref[...]
    h_ref[...] = h
    p = p_ref[...]
    s_ref[...] = _dot(h, p) / _pnorm(p)


def _a0s2_body(a_ref, ohf_ref, ohb_ref, o_ref):
    o_ref[...] = _dotb(a_ref[...], ohf_ref[...]) + ohb_ref[...]


def _kA_body(oht_ref, h2_ref, a0s2_ref, u1_ref, u2_ref, gm2_ref,
             w3_ref, b3_ref, p3_ref,
             h3c_ref, a2c_ref, u1c_ref, u2c_ref, s3c_ref):
    oht = oht_ref[...]
    u1c = _dot(oht, u1_ref[...])
    u2c = _dot(oht, u2_ref[...])
    gm2c = _dot(oht, gm2_ref[...])
    u1c_ref[...] = u1c
    u2c_ref[...] = u2c
    h2c = _dotb(oht, h2_ref[...])
    a2cp = _dotb(oht, a0s2_ref[...])
    a2c_ref[...] = a2cp
    t3c = u2c * _dotb(gm2c * h2c, w3_ref[...])
    h3c = u2c * _dotb(a2cp, t3c) + b3_ref[...]
    h3c_ref[...] = h3c
    p = p3_ref[...]
    s3c_ref[...] = _dot(h3c, p) / _pnorm(p)


def _kB_body(s3c_ref, s3ct_ref, h3c_ref, a2c_ref, u1c_ref, u2c_ref,
             ce0_ref, te0_ref, ce1_ref, te1_ref,
             wu1_ref, bu1_ref, wu2_ref,
             tu2c_ref):
    s = s3c_ref[...]
    st = s3ct_ref[...]
    ridx = jax.lax.broadcasted_iota(jnp.int32, (K2, 1), 0)
    cidx = jax.lax.broadcasted_iota(jnp.int32, (1, K2), 1)
    gt = (st > s) | ((st == s) & (cidx < ridx))
    r = jnp.sum(gt.astype(jnp.float32), axis=1, keepdims=True)
    m3c = (r < 6).astype(jnp.float32)
    gm3c = m3c * jnp.tanh(s)
    u2c = u2c_ref[...]
    x2uc = gm3c * h3c_ref[...] * ce0_ref[...] + m3c * te0_ref[...]
    tu1c = u2c * _dotb(x2uc, wu1_ref[...])
    x2rc = u2c * _dotb(a2c_ref[...], tu1c) + bu1_ref[...]
    x1uc = x2rc * ce1_ref[...] + te1_ref[...]
    tu2c_ref[...] = u1c_ref[...] * _dotb(x1uc, wu2_ref[...])


def _gcnu2_body(a0s2_ref, tu2c_ref, u1_ref, bu2_ref, o_ref):
    o_ref[...] = u1_ref[...] * _dotb(a0s2_ref[...], tu2c_ref[...]) + bu2_ref[...]


def _y_body(m1_ref, x1r_ref, ce2_ref, te2_ref, wu3_ref, o_ref):
    xu = m1_ref[...] * (x1r_ref[...] * ce2_ref[...] + te2_ref[...])
    o_ref[...] = _dotb(xu, wu3_ref[...]).astype(jnp.bfloat16)


def _fin_body(n0_ref, y_ref, b_ref, o_ref):
    o_ref[...] = _dotb(n0_ref[...], y_ref[...]) + b_ref[...]


def _vspec(bm):
    return pl.BlockSpec((bm, 1), lambda i: (i, 0))


def _fix(shape):
    return pl.BlockSpec(shape, lambda i: (0, 0))


def _rowspec(bm, n):
    return pl.BlockSpec((bm, n), lambda i: (i, 0))


def _sds(shape, dtype=jnp.float32):
    return jax.ShapeDtypeStruct(shape, dtype)


def _run_emb(cbp, cm, t, ws):
    dims = [F3, F3, F2, F2, F1, F1]
    outs = tuple(_sds((1, d)) for d in dims)
    return pl.pallas_call(_emb_body, out_shape=outs)(cbp, cm, t, *ws)


def _run_dinv(a):
    return pl.pallas_call(
        _dinv_body, grid=(NBLK,),
        in_specs=[_rowspec(BR, N)],
        out_specs=_vspec(BR),
        out_shape=_sds((N, 1)),
    )(a)


def _run_n0h1(a, dvr, dv, w1, b1, p1):
    return pl.pallas_call(
        _n0h1_body, grid=(NBLK,),
        in_specs=[_rowspec(BR, N), _fix((1, N)), _vspec(BR),
                  _fix((N, F1)), _fix((1, F1)), _fix((F1, 1))],
        out_specs=(_rowspec(BR, N), _rowspec(BR, F1), _vspec(BR)),
        out_shape=(_sds((N, N)), _sds((N, F1)), _sds((N, 1))),
    )(a, dvr, dv, w1, b1, p1)


def _run_rank1(s):
    return pl.pallas_call(
        _rank1_body,
        out_shape=(_sds((N, 1)), _sds((N, 1))),
    )(s, s.reshape(1, N))


def _run_rank2(s, mprev):
    return pl.pallas_call(
        _rank2_body,
        out_shape=(_sds((N, 1)), _sds((N, 1)), _sds((N, K2)), _sds((K2, N))),
    )(s, s.reshape(1, N), mprev, mprev.reshape(1, N))


def _run_deg(a, m):
    return pl.pallas_call(
        _deg_body, grid=(NBLK,),
        in_specs=[_rowspec(BR, N), _fix((N, 1)), _vspec(BR)],
        out_specs=_vspec(BR),
        out_shape=_sds((N, 1)),
    )(a, m, m)


def _run_xw(avec, h, u, w):
    fi, fo = w.shape
    return pl.pallas_call(
        _xw_body, grid=(NBLK,),
        in_specs=[_vspec(BR), _rowspec(BR, fi), _vspec(BR), _fix((fi, fo))],
        out_specs=_rowspec(BR, fo),
        out_shape=_sds((N, fo)),
    )(avec, h, u, w)


def _run_gcn(a, tmat, u, b, p):
    f = tmat.shape[1]
    return pl.pallas_call(
        _gcn_body, grid=(NBLK,),
        in_specs=[_rowspec(BR, N), _fix((N, f)), _rowspec(BR, f),
                  _vspec(BR), _fix((1, f)), _fix((f, 1))],
        out_specs=(_rowspec(BR, f), _vspec(BR)),
        out_shape=(_sds((N, f)), _sds((N, 1))),
    )(a, tmat, tmat, u, b, p)


def _run_a0s2(a, oh):
    return pl.pallas_call(
        _a0s2_body, grid=(NBLK,),
        in_specs=[_rowspec(BR, N), _fix((N, K2)), _rowspec(BR, K2)],
        out_specs=_rowspec(BR, K2),
        out_shape=_sds((N, K2)),
    )(a, oh, oh)


def _run_kA(oht, h2, a0s2, u1, u2, gm2, w3, b3, p3):
    return pl.pallas_call(
        _kA_body,
        out_shape=(_sds((K2, F3)), _sds((K2, K2)), _sds((K2, 1)),
                   _sds((K2, 1)), _sds((K2, 1))),
    )(oht, h2, a0s2, u1, u2, gm2, w3, b3, p3)


def _run_kB(s3c, h3c, a2c, u1c, u2c, ce0, te0, ce1, te1, wu1, bu1, wu2):
    return pl.pallas_call(
        _kB_body,
        out_shape=_sds((K2, F1)),
    )(s3c, s3c.reshape(1, K2), h3c, a2c, u1c, u2c,
      ce0, te0, ce1, te1, wu1, bu1, wu2)


def _run_gcnu2(a0s2, tu2c, u1, bu2):
    return pl.pallas_call(
        _gcnu2_body, grid=(NBLK,),
        in_specs=[_rowspec(BR, K2), _fix((K2, F1)), _vspec(BR), _fix((1, F1))],
        out_specs=_rowspec(BR, F1),
        out_shape=_sds((N, F1)),
    )(a0s2, tu2c, u1, bu2)


def _run_y(m1, x1r, ce2, te2, wu3):
    return pl.pallas_call(
        _y_body, grid=(NBLK,),
        in_specs=[_vspec(BR), _rowspec(BR, F1), _fix((1, F1)), _fix((1, F1)),
                  _fix((F1, N))],
        out_specs=_rowspec(BR, N),
        out_shape=_sds((N, N), jnp.bfloat16),
    )(m1, x1r, ce2, te2, wu3)


def _run_final(n0, y, b):
    bm = 256
    return pl.pallas_call(
        _fin_body, grid=(N // bm,),
        in_specs=[_rowspec(bm, N), _fix((N, N)), _fix((1, N))],
        out_specs=_rowspec(bm, N),
        out_shape=_sds((N, N)),
    )(n0, y, b)


def kernel(x, c, t, context_mask, W1, b1, p1, W2, b2, p2, W3, b3, p3,
           Wu1, bu1, Wu2, bu2, Wu3, bu3,
           TW0a, Tb0a, TW0b, Tb0b, CW0a, Cb0a, CW0b, Cb0b,
           TW1a, Tb1a, TW1b, Tb1b, CW1a, Cb1a, CW1b, Cb1b,
           TW2a, Tb2a, TW2b, Tb2b, CW2a, Cb2a, CW2b, Cb2b):
    a0 = x[0, 0]

    w1p = _pad2(W1, N, F1)
    b1p = _pad_row(b1, F1)
    p1p = _pad_col(p1, F1)
    w2p = _pad2(W2, F1, F2)
    b2p = _pad_row(b2, F2)
    p2p = _pad_col(p2, F2)
    w3p = _pad2(W3, F2, F3)
    b3p = _pad_row(b3, F3)
    p3p = _pad_col(p3, F3)
    wu1p = _pad2(Wu1, F3, F2)
    bu1p = _pad_row(bu1, F2)
    wu2p = _pad2(Wu2, F2, F1)
    bu2p = _pad_row(bu2, F1)
    wu3p = _pad2(Wu3, F1, N)
    bu3p = _pad_row(bu3, N)

    cbp = _pad2(c, 1, 16)
    cmp_ = context_mask.reshape(1, 1)
    tp = t.reshape(1, 1)
    embw = [
        _pad2(TW0a, 1, F3), _pad_row(Tb0a, F3), _pad2(TW0b, F3, F3), _pad_row(Tb0b, F3),
        _pad2(CW0a, 16, F3), _pad_row(Cb0a, F3), _pad2(CW0b, F3, F3), _pad_row(Cb0b, F3),
        _pad2(TW1a, 1, F2), _pad_row(Tb1a, F2), _pad2(TW1b, F2, F2), _pad_row(Tb1b, F2),
        _pad2(CW1a, 16, F2), _pad_row(Cb1a, F2), _pad2(CW1b, F2, F2), _pad_row(Cb1b, F2),
        _pad2(TW2a, 1, F1), _pad_row(Tb2a, F1), _pad2(TW2b, F1, F1), _pad_row(Tb2b, F1),
        _pad2(CW2a, 16, F1), _pad_row(Cb2a, F1), _pad2(CW2b, F1, F1), _pad_row(Cb2b, F1),
    ]
    temb0, cemb0, temb1, cemb1, temb2, cemb2 = _run_emb(cbp, cmp_, tp, embw)

    dv = _run_dinv(a0)
    n0, h1, s1 = _run_n0h1(a0, dv.reshape(1, N), dv, w1p, b1p, p1p)

    m1, gm1 = _run_rank1(s1)
    u1 = _run_deg(a0, m1)
    t2 = _run_xw(gm1, h1, u1, w2p)
    h2, s2 = _run_gcn(a0, t2, u1, b2p, p2p)

    m2, gm2, oh2, oh2t = _run_rank2(s2, m1)
    u2 = _run_deg(a0, m2)
    a0s2 = _run_a0s2(a0, oh2)
    h3c, a2c, u1c, u2c, s3c = _run_kA(oh2t, h2, a0s2, u1, u2, gm2,
                                      w3p, b3p, p3p)
    tu2c = _run_kB(s3c, h3c, a2c, u1c, u2c, cemb0, temb0, cemb1, temb1,
                   wu1p, bu1p, wu2p)
    x1r = _run_gcnu2(a0s2, tu2c, u1, bu2p)

    y = _run_y(m1, x1r, cemb2, temb2, wu3p)
    return _run_final(n0, y, bu3p)
